# baseline (speedup 1.0000x reference)
import functools

import jax
import jax.numpy as jnp
from jax import lax
from jax.experimental import pallas as pl
from jax.experimental.pallas import tpu as pltpu
from jax.experimental.pallas import tpu_sc as plsc

_B = 4096
_L = 50
_D_NEWS = 400
_CAT_NUM = 20
_SUBCAT_NUM = 300
_CAT_DIM = 100
_SUBCAT_DIM = 100
_D_EMB = _CAT_DIM + _SUBCAT_DIM
_D_GATHER = 256
_D_OUT = _D_NEWS + _D_EMB
_N_FUSED = _CAT_NUM * _SUBCAT_NUM

_NUM_CORES = 2
_NUM_SUBCORES = 16
_NW = _NUM_CORES * _NUM_SUBCORES
_B_PER_W = _B // _NW
_L_PAD = 56

_ROW_BLOCK = 16


def _fuse_tables_tc(cat_tab, sub_tab):
    def body(cat_ref, sub_ref, out_ref):
        cat = cat_ref[...]
        sub = sub_ref[...]
        cat_rep = lax.broadcast_in_dim(
            cat, (_CAT_NUM, _SUBCAT_NUM, _CAT_DIM), (0, 2)
        ).reshape(_N_FUSED, _CAT_DIM)
        sub_rep = lax.broadcast_in_dim(
            sub, (_CAT_NUM, _SUBCAT_NUM, _SUBCAT_DIM), (1, 2)
        ).reshape(_N_FUSED, _SUBCAT_DIM)
        pad = jnp.zeros((_N_FUSED, _D_GATHER - _D_EMB), jnp.float32)
        out_ref[...] = jnp.concatenate([cat_rep, sub_rep, pad], axis=1)

    return pl.pallas_call(
        body,
        out_shape=jax.ShapeDtypeStruct((_N_FUSED, _D_GATHER), jnp.float32),
    )(cat_tab, sub_tab)


def _make_sc_gather():
    mesh = plsc.VectorSubcoreMesh(core_axis_name="c", subcore_axis_name="s")

    @functools.partial(
        pl.kernel,
        mesh=mesh,
        out_type=jax.ShapeDtypeStruct((_B, _L_PAD, _D_GATHER), jnp.float32),
        scratch_types=[
            pltpu.VMEM((_B_PER_W, _L_PAD), jnp.int32),
            pltpu.VMEM((_L_PAD, _D_GATHER), jnp.float32),
            pltpu.VMEM((_L_PAD, _D_GATHER), jnp.float32),
            pltpu.SemaphoreType.DMA,
            pltpu.SemaphoreType.DMA,
            pltpu.SemaphoreType.DMA,
            pltpu.SemaphoreType.DMA,
        ],
    )
    def sc_gather(fidx_hbm, fused_tab_hbm, emb_hbm,
                  fidx_v, buf0_v, buf1_v, sem_g0, sem_g1, sem_w0, sem_w1):
        wid = lax.axis_index("s") * _NUM_CORES + lax.axis_index("c")
        base0 = wid * _B_PER_W
        pltpu.sync_copy(fidx_hbm.at[pl.ds(base0, _B_PER_W)], fidx_v)

        def gather(j, buf, sem):
            return pltpu.async_copy(fused_tab_hbm.at[fidx_v.at[j]], buf, sem)

        def wb(j, buf, sem):
            return pltpu.async_copy(buf, emb_hbm.at[base0 + j], sem)

        def loop_body(j, carry):
            gather(j, buf0_v, sem_g0).wait()
            wb(j, buf0_v, sem_w0).wait()
            return carry

        lax.fori_loop(0, _B_PER_W, loop_body, 0)

    return sc_gather


_SC_GATHER = _make_sc_gather()


def _concat_tc(news3d, emb):
    def body(news_ref, emb_ref, out_ref):
        out_ref[...] = jnp.concatenate(
            [news_ref[...], emb_ref[:, : _L, : _D_EMB]], axis=2)

    grid = (_B // _ROW_BLOCK,)
    return pl.pallas_call(
        body,
        grid=grid,
        in_specs=[
            pl.BlockSpec((_ROW_BLOCK, _L, _D_NEWS), lambda i: (i, 0, 0)),
            pl.BlockSpec((_ROW_BLOCK, _L_PAD, _D_GATHER), lambda i: (i, 0, 0)),
        ],
        out_specs=pl.BlockSpec((_ROW_BLOCK, _L, _D_OUT), lambda i: (i, 0, 0)),
        out_shape=jax.ShapeDtypeStruct((_B, _L, _D_OUT), jnp.float32),
    )(news3d, emb)


def kernel(news_representation, category, subCategory, category_table,
           subCategory_table):
    cat = category.astype(jnp.int32)
    sub = subCategory.astype(jnp.int32)
    fidx = cat * _SUBCAT_NUM + sub
    fidx = jnp.pad(fidx, ((0, 0), (0, _L_PAD - _L)))
    fused_tab = _fuse_tables_tc(category_table, subCategory_table)
    emb = _SC_GATHER(fidx, fused_tab)
    return _concat_tc(news_representation, emb)
